# Initial kernel scaffold; baseline (speedup 1.0000x reference)
import jax
import jax.numpy as jnp
from jax.experimental import pallas as pl


def kernel(x, pos_table, token_table):
    raise NotImplementedError("write your pallas kernel here")


# speedup vs baseline: 7.6383x; 7.6383x over previous
import functools

import jax
import jax.numpy as jnp
from jax import lax
from jax.experimental import pallas as pl
from jax.experimental.pallas import tpu as pltpu
from jax.experimental.pallas import tpu_sc as plsc

_NUM_CORES = 2
_NUM_SUBCORES = 16
_NW = _NUM_CORES * _NUM_SUBCORES
_LANES = 16


def _emb_kernel(B, S, D, V):
    b_per_w = B // _NW
    s_half = S // 2
    mesh = plsc.VectorSubcoreMesh(
        core_axis_name="c", subcore_axis_name="s",
        num_cores=_NUM_CORES, num_subcores=_NUM_SUBCORES)

    @functools.partial(
        pl.kernel,
        out_type=jax.ShapeDtypeStruct((B, S, D), jnp.float32),
        mesh=mesh,
        scratch_types=[
            pltpu.VMEM((2, s_half), jnp.int32),
            pltpu.VMEM((S, D), jnp.float32),
            pltpu.VMEM((S, D), jnp.float32),
            pltpu.SemaphoreType.DMA,
        ],
    )
    def body(x_hbm, pos_hbm, tok_hbm, out_hbm, idx_v, pos_v, buf, sem):
        wid = lax.axis_index("s") * _NUM_CORES + lax.axis_index("c")
        pltpu.sync_copy(pos_hbm, pos_v)

        def one_batch(i, carry):
            b = wid * b_per_w + i
            pltpu.sync_copy(x_hbm.at[b], idx_v)
            cp0 = pltpu.async_copy(
                tok_hbm.at[idx_v.at[0]], buf.at[pl.ds(0, s_half)], sem)
            cp1 = pltpu.async_copy(
                tok_hbm.at[idx_v.at[1]], buf.at[pl.ds(s_half, s_half)], sem)
            cp0.wait()
            cp1.wait()

            def add_row(r, c2):
                for c in range(D // _LANES):
                    sl = pl.ds(c * _LANES, _LANES)
                    buf[r, sl] = buf[r, sl] + pos_v[r, sl]
                return c2

            lax.fori_loop(0, S, add_row, 0, unroll=False)
            pltpu.sync_copy(buf, out_hbm.at[b])
            return carry

        lax.fori_loop(0, b_per_w, one_batch, 0, unroll=False)

    return body


def kernel(x, pos_table, token_table):
    B, S = x.shape
    V, D = token_table.shape
    x3 = x.astype(jnp.int32).reshape(B, 2, S // 2)
    out = _emb_kernel(B, S, D, V)(x3, pos_table, token_table)
    return out


# speedup vs baseline: 12.5069x; 1.6374x over previous
import functools

import jax
import jax.numpy as jnp
from jax import lax
from jax.experimental import pallas as pl
from jax.experimental.pallas import tpu as pltpu
from jax.experimental.pallas import tpu_sc as plsc

_NUM_CORES = 2
_NUM_SUBCORES = 16
_NW = _NUM_CORES * _NUM_SUBCORES
_LANES = 16


def _emb_kernel(B, S, D, V):
    b_per_w = B // _NW
    s_half = S // 2
    half = b_per_w // 2
    mesh = plsc.VectorSubcoreMesh(
        core_axis_name="c", subcore_axis_name="s",
        num_cores=_NUM_CORES, num_subcores=_NUM_SUBCORES)

    @functools.partial(
        pl.kernel,
        out_type=jax.ShapeDtypeStruct((B, S, D), jnp.float32),
        mesh=mesh,
        scratch_types=[
            pltpu.VMEM((b_per_w, 2, s_half), jnp.int32),
            pltpu.VMEM((S, D), jnp.float32),
            pltpu.VMEM((S, D), jnp.float32),
            pltpu.VMEM((S, D), jnp.float32),
            pltpu.SemaphoreType.DMA,
            pltpu.SemaphoreType.DMA,
            pltpu.SemaphoreType.DMA,
            pltpu.SemaphoreType.DMA,
        ],
    )
    def body(x_hbm, pos_hbm, tok_hbm, out_hbm,
             idx_v, pos_v, buf0, buf1, g0, g1, o0, o1):
        wid = lax.axis_index("s") * _NUM_CORES + lax.axis_index("c")
        base = wid * b_per_w
        bufs, gsems, osems = (buf0, buf1), (g0, g1), (o0, o1)

        pltpu.sync_copy(x_hbm.at[pl.ds(base, b_per_w)], idx_v)
        pltpu.sync_copy(pos_hbm, pos_v)

        def gather(i, buf, sem):
            for j in range(2):
                pltpu.async_copy(
                    tok_hbm.at[idx_v.at[i, j]],
                    buf.at[pl.ds(j * s_half, s_half)], sem)

        def gather_wait(buf, sem):
            for j in range(2):
                pltpu.make_async_copy(
                    tok_hbm.at[idx_v.at[0, j]],
                    buf.at[pl.ds(j * s_half, s_half)], sem).wait()

        def out_wait(buf, sem):
            pltpu.make_async_copy(buf, out_hbm.at[base], sem).wait()

        def add_pos(buf):
            def add_row(r, c):
                for ch in range(D // _LANES):
                    sl = pl.ds(ch * _LANES, _LANES)
                    buf[r, sl] = buf[r, sl] + pos_v[r, sl]
                return c
            lax.fori_loop(0, S, add_row, 0, unroll=False)

        gather(0, buf0, g0)

        def step(g, carry):
            for k in range(2):
                i = 2 * g + k
                gather_wait(bufs[k], gsems[k])
                if k == 0:
                    @pl.when(g >= 1)
                    def _():
                        out_wait(bufs[1], osems[1])
                    gather(i + 1, bufs[1], gsems[1])
                else:
                    out_wait(bufs[0], osems[0])
                    @pl.when(g < half - 1)
                    def _():
                        gather(i + 1, bufs[0], gsems[0])
                add_pos(bufs[k])
                pltpu.async_copy(bufs[k], out_hbm.at[base + i], osems[k])
            return carry

        lax.fori_loop(0, half, step, 0, unroll=False)
        out_wait(buf1, o1)

    return body


def kernel(x, pos_table, token_table):
    B, S = x.shape
    V, D = token_table.shape
    x3 = x.astype(jnp.int32).reshape(B, 2, S // 2)
    out = _emb_kernel(B, S, D, V)(x3, pos_table, token_table)
    return out
